# baseline (speedup 1.0000x reference)
import functools

import jax
import jax.numpy as jnp
from jax import lax
from jax.experimental import pallas as pl
from jax.experimental.pallas import tpu as pltpu
from jax.experimental.pallas import tpu_sc as plsc

_B = 16384
_C = 20
_NEG = 16
_D = 64
_K = 1 + _NEG
_L = 16
_ND = _D // _L

_NC = 2
_NS = 16
_NW = _NC * _NS
_BPW = _B // _NW
_CB = 32
_NCHUNK = _BPW // _CB
_G_CTX = (_CB * _C) // 128
_G_NEG = (_CB * _NEG) // 128


def _sc_body(cen_hbm, ctx_hbm, neg_hbm, wc_hbm, wx_hbm, out_hbm,
             cen_idx, ctx_idx, neg_idx, cen_rows, ctx_rows, neg_rows,
             scores_v, sem):
    wid = lax.axis_index("s") * _NC + lax.axis_index("c")
    base = wid * _BPW
    lanes = jnp.arange(_L, dtype=jnp.int32)

    def chunk_body(i, carry):
        b0 = base + i * _CB
        pltpu.sync_copy(cen_hbm.at[pl.ds(b0, _CB)], cen_idx)
        pltpu.sync_copy(ctx_hbm.at[pl.ds(b0 * _C, _CB * _C)], ctx_idx)
        pltpu.sync_copy(neg_hbm.at[pl.ds(b0 * _NEG, _CB * _NEG)], neg_idx)
        cps = []
        for g in range(_G_CTX):
            cps.append(pltpu.async_copy(
                wx_hbm.at[ctx_idx.at[pl.ds(g * 128, 128)]],
                ctx_rows.at[pl.ds(g * 128, 128)], sem))
        for g in range(_G_NEG):
            cps.append(pltpu.async_copy(
                wc_hbm.at[neg_idx.at[pl.ds(g * 128, 128)]],
                neg_rows.at[pl.ds(g * 128, 128)], sem))
        cps.append(pltpu.async_copy(wc_hbm.at[cen_idx], cen_rows, sem))
        for cp in cps:
            cp.wait()

        for bb in range(_CB // _L):
            def b_body(t, rows):
                b = bb * _L + t
                lane_mask = lanes == t
                accs = [jnp.zeros((_L,), jnp.float32) for _ in range(_ND)]
                for c in range(_C):
                    r = b * _C + c
                    for d in range(_ND):
                        accs[d] = accs[d] + ctx_rows[r, pl.ds(d * _L, _L)]

                def dot(rref, r):
                    p = rref[r, pl.ds(0, _L)] * accs[0]
                    for d in range(1, _ND):
                        p = p + rref[r, pl.ds(d * _L, _L)] * accs[d]
                    return jnp.sum(p)

                new_rows = [jnp.where(lane_mask, dot(cen_rows, b), rows[0])]
                for j in range(_NEG):
                    new_rows.append(jnp.where(
                        lane_mask, -dot(neg_rows, b * _NEG + j), rows[1 + j]))
                return tuple(new_rows)

            zeros = jnp.zeros((_L,), jnp.float32)
            rows = lax.fori_loop(0, _L, b_body, (zeros,) * _K)
            off = i * _CB + bb * _L
            for k in range(_K):
                scores_v[k, pl.ds(off, _L)] = rows[k]
        return carry

    lax.fori_loop(0, _NCHUNK, chunk_body, 0)
    pltpu.sync_copy(scores_v, out_hbm.at[wid])


_sc_scores = functools.partial(
    pl.kernel,
    mesh=plsc.VectorSubcoreMesh(core_axis_name="c", subcore_axis_name="s"),
    compiler_params=pltpu.CompilerParams(
        needs_layout_passes=False, use_tc_tiling_on_sc=False),
    out_type=jax.ShapeDtypeStruct((_NW, _K, _BPW), jnp.float32),
    scratch_types=[
        pltpu.VMEM((_CB,), jnp.int32),
        pltpu.VMEM((_CB * _C,), jnp.int32),
        pltpu.VMEM((_CB * _NEG,), jnp.int32),
        pltpu.VMEM((_CB, _D), jnp.float32),
        pltpu.VMEM((_CB * _C, _D), jnp.float32),
        pltpu.VMEM((_CB * _NEG, _D), jnp.float32),
        pltpu.VMEM((_K, _BPW), jnp.float32),
        pltpu.SemaphoreType.DMA,
    ],
)(_sc_body)


def _loss_body(s_ref, o_ref):
    x = s_ref[...]
    ls = jnp.minimum(x, 0.0) - jnp.log1p(jnp.exp(-jnp.abs(x)))
    o_ref[0, 0] = -jnp.sum(ls) * (1.0 / (_B * _K))


_loss = pl.pallas_call(
    _loss_body,
    out_shape=jax.ShapeDtypeStruct((1, 1), jnp.float32),
    out_specs=pl.BlockSpec(memory_space=pltpu.SMEM),
)


def kernel(center, contexts, negatives, W_center, W_context):
    cen = center.astype(jnp.int32)
    ctx = contexts.astype(jnp.int32).reshape(-1)
    neg = negatives.astype(jnp.int32).reshape(-1)
    scores = _sc_scores(cen, ctx, neg, W_center, W_context)
    return _loss(scores)[0, 0]
